# Initial kernel scaffold; baseline (speedup 1.0000x reference)
import jax
import jax.numpy as jnp
from jax.experimental import pallas as pl


def kernel(habitat, substrate, month, hour, camera_model, camera_maker, latitude, longitude, habitat_table, substrate_table, month_table, hour_table, camera_model_table, camera_maker_table, W1, b1, W2, b2):
    raise NotImplementedError("write your pallas kernel here")


# speedup vs baseline: 1.2650x; 1.2650x over previous
import functools

import jax
import jax.numpy as jnp
from jax import lax
from jax.experimental import pallas as pl
from jax.experimental.pallas import tpu as pltpu
from jax.experimental.pallas import tpu_sc as plsc

B = 16384
D = 64
NT = 6
CHUNK = 128


def _mlp_body(lat_ref, lon_ref, w1_ref, b1_ref, w2_ref, b2_ref, o_ref):
    h = jnp.maximum(
        lat_ref[...] * w1_ref[0:1, :] + lon_ref[...] * w1_ref[1:2, :] + b1_ref[...],
        0.0,
    )
    o_ref[...] = jnp.dot(h, w2_ref[...], preferred_element_type=jnp.float32) + b2_ref[...]


def _mlp(latitude, longitude, W1, b1, W2, b2):
    return pl.pallas_call(
        _mlp_body,
        out_shape=jax.ShapeDtypeStruct((B, D), jnp.float32),
    )(
        latitude.reshape(B, 1),
        longitude.reshape(B, 1),
        W1,
        b1.reshape(1, 32),
        W2,
        b2.reshape(1, D),
    )


def _sc_embed(h_i, s_i, m_i, hr_i, cmod_i, cmak_i, g,
              h_t, s_t, m_t, hr_t, cmod_t, cmak_t):
    info = plsc.get_sparse_core_info()
    NC, NS = info.num_cores, info.num_subcores
    NW = NC * NS
    b_per_w = B // NW
    n_sub = b_per_w // CHUNK

    mesh = plsc.VectorSubcoreMesh(core_axis_name="c", subcore_axis_name="s")

    @functools.partial(
        pl.kernel,
        mesh=mesh,
        out_type=jax.ShapeDtypeStruct((B, (NT + 1) * D), jnp.float32),
        scratch_types=[
            pltpu.VMEM((CHUNK,), jnp.int32),
            pltpu.VMEM((CHUNK, D), jnp.float32),
            pltpu.SemaphoreType.DMA,
        ],
        compiler_params=pltpu.CompilerParams(use_tc_tiling_on_sc=False),
    )
    def k(h_ref, s_ref, m_ref, hr_ref, cmod_ref, cmak_ref, g_ref,
          ht_ref, st_ref, mt_ref, hrt_ref, cmodt_ref, cmakt_ref,
          out_ref, idx_v, rows_v, sem):
        wid = lax.axis_index("s") * NC + lax.axis_index("c")
        base = wid * b_per_w
        srcs = [(h_ref, ht_ref), (s_ref, st_ref), (m_ref, mt_ref),
                (hr_ref, hrt_ref), (cmod_ref, cmodt_ref), (cmak_ref, cmakt_ref)]
        for c in range(n_sub):
            off = base + c * CHUNK
            for t, (iref, tref) in enumerate(srcs):
                pltpu.sync_copy(iref.at[pl.ds(off, CHUNK)], idx_v)
                pltpu.async_copy(tref.at[idx_v], rows_v, sem).wait()
                pltpu.sync_copy(rows_v, out_ref.at[pl.ds(off, CHUNK), pl.ds(t * D, D)])
            pltpu.sync_copy(g_ref.at[pl.ds(off, CHUNK)], rows_v)
            pltpu.sync_copy(rows_v, out_ref.at[pl.ds(off, CHUNK), pl.ds(NT * D, D)])

    return k(h_i, s_i, m_i, hr_i, cmod_i, cmak_i, g,
             h_t, s_t, m_t, hr_t, cmod_t, cmak_t)


def kernel(habitat, substrate, month, hour, camera_model, camera_maker,
           latitude, longitude,
           habitat_table, substrate_table, month_table, hour_table,
           camera_model_table, camera_maker_table, W1, b1, W2, b2):
    g = _mlp(latitude, longitude, W1, b1, W2, b2)
    idx = [x.astype(jnp.int32) for x in
           (habitat, substrate, month, hour, camera_model, camera_maker)]
    return _sc_embed(*idx, g,
                     habitat_table, substrate_table, month_table, hour_table,
                     camera_model_table, camera_maker_table)


# speedup vs baseline: 1.3295x; 1.0509x over previous
import functools

import jax
import jax.numpy as jnp
from jax import lax
from jax.experimental import pallas as pl
from jax.experimental.pallas import tpu as pltpu
from jax.experimental.pallas import tpu_sc as plsc

B = 16384
D = 64
NT = 6
CHUNK = 128


def _mlp_body(lat_ref, lon_ref, w1_ref, b1_ref, w2_ref, b2_ref, o_ref):
    h = jnp.maximum(
        lat_ref[...] * w1_ref[0:1, :] + lon_ref[...] * w1_ref[1:2, :] + b1_ref[...],
        0.0,
    )
    o_ref[...] = jnp.dot(h, w2_ref[...], preferred_element_type=jnp.float32) + b2_ref[...]


def _mlp(latitude, longitude, W1, b1, W2, b2):
    return pl.pallas_call(
        _mlp_body,
        out_shape=jax.ShapeDtypeStruct((B, D), jnp.float32),
    )(
        latitude.reshape(B, 1),
        longitude.reshape(B, 1),
        W1,
        b1.reshape(1, 32),
        W2,
        b2.reshape(1, D),
    )


def _sc_embed(h_i, s_i, m_i, hr_i, cmod_i, cmak_i, g,
              h_t, s_t, m_t, hr_t, cmod_t, cmak_t):
    info = plsc.get_sparse_core_info()
    NC, NS = info.num_cores, info.num_subcores
    NW = NC * NS
    b_per_w = B // NW
    n_sub = b_per_w // CHUNK

    mesh = plsc.VectorSubcoreMesh(core_axis_name="c", subcore_axis_name="s")

    @functools.partial(
        pl.kernel,
        mesh=mesh,
        out_type=jax.ShapeDtypeStruct((B, (NT + 1) * D), jnp.float32),
        scratch_types=[
            pltpu.VMEM((NT, b_per_w), jnp.int32),
            pltpu.VMEM((2, NT + 1, CHUNK, D), jnp.float32),
            pltpu.SemaphoreType.DMA,
            pltpu.SemaphoreType.DMA,
            pltpu.SemaphoreType.DMA,
        ],
        compiler_params=pltpu.CompilerParams(use_tc_tiling_on_sc=False),
    )
    def k(h_ref, s_ref, m_ref, hr_ref, cmod_ref, cmak_ref, g_ref,
          ht_ref, st_ref, mt_ref, hrt_ref, cmodt_ref, cmakt_ref,
          out_ref, idx_v, bufs, sem_g, sem_w0, sem_w1):
        wid = lax.axis_index("s") * NC + lax.axis_index("c")
        base = wid * b_per_w
        idx_refs = [h_ref, s_ref, m_ref, hr_ref, cmod_ref, cmak_ref]
        tbl_refs = [ht_ref, st_ref, mt_ref, hrt_ref, cmodt_ref, cmakt_ref]
        sem_w = [sem_w0, sem_w1]
        for t in range(NT):
            pltpu.sync_copy(idx_refs[t].at[pl.ds(base, b_per_w)], idx_v.at[t])
        writes = {0: [], 1: []}
        for c in range(n_sub):
            p = c % 2
            for wdesc in writes[p]:
                wdesc.wait()
            writes[p] = []
            off = base + c * CHUNK
            gathers = []
            for t in range(NT):
                gathers.append(pltpu.async_copy(
                    tbl_refs[t].at[idx_v.at[t, pl.ds(c * CHUNK, CHUNK)]],
                    bufs.at[p, t], sem_g))
            gathers.append(pltpu.async_copy(
                g_ref.at[pl.ds(off, CHUNK)], bufs.at[p, NT], sem_g))
            for gd in gathers:
                gd.wait()
            for t in range(NT + 1):
                writes[p].append(pltpu.async_copy(
                    bufs.at[p, t],
                    out_ref.at[pl.ds(off, CHUNK), pl.ds(t * D, D)],
                    sem_w[p]))
        for p in (0, 1):
            for wdesc in writes[p]:
                wdesc.wait()

    return k(h_i, s_i, m_i, hr_i, cmod_i, cmak_i, g,
             h_t, s_t, m_t, hr_t, cmod_t, cmak_t)


def kernel(habitat, substrate, month, hour, camera_model, camera_maker,
           latitude, longitude,
           habitat_table, substrate_table, month_table, hour_table,
           camera_model_table, camera_maker_table, W1, b1, W2, b2):
    g = _mlp(latitude, longitude, W1, b1, W2, b2)
    idx = [x.astype(jnp.int32) for x in
           (habitat, substrate, month, hour, camera_model, camera_maker)]
    return _sc_embed(*idx, g,
                     habitat_table, substrate_table, month_table, hour_table,
                     camera_model_table, camera_maker_table)
